# baseline (speedup 1.0000x reference)
import functools

import jax
import jax.numpy as jnp
from jax import lax
from jax.experimental import pallas as pl
from jax.experimental.pallas import tpu as pltpu
from jax.experimental.pallas import tpu_sc as plsc

N_NODES = 10000
N_EDGES = 160000
IN_DIM = 256
HID_DIM = 512

CW = 16
NC = 2
NS = 16
EDGES_PER_TILE = N_EDGES // (NC * NS)
CHUNK = 125
NCHUNK = EDGES_PER_TILE // CHUNK
ROWS_PER_TILE = N_NODES // NS
QSCALE = 256.0


def _sc_scatter(emb_q, ed, ones_h, zf, zc, feat, cnt,
                idx_a, idx_b, rows_a, rows_b, ones_v, shf, shc,
                sem_ga, sem_gb, sem_sa, sem_sb, sem_oa, sem_ob):
    c = lax.axis_index("c")
    s = lax.axis_index("s")
    pltpu.sync_copy(zf, shf.at[pl.ds(s * ROWS_PER_TILE, ROWS_PER_TILE)])
    pltpu.sync_copy(zc, shc.at[pl.ds(s * ROWS_PER_TILE, ROWS_PER_TILE)])
    pltpu.sync_copy(ones_h, ones_v)
    plsc.subcore_barrier()

    def ones_scatter(jj, idx_v, sem_o):
        @pl.when(jj > 0)
        def _():
            pltpu.make_async_copy(ones_h, ones_v, sem_o).wait()
        pltpu.async_copy(ones_v, shc.at[idx_v.at[0]], sem_o, add=True)

    NJJ = NCHUNK // 2
    pltpu.sync_copy(ed.at[c, s, 0], idx_a)
    pltpu.async_copy(emb_q.at[idx_a.at[1]], rows_a, sem_ga)

    def body(jj, carry):
        a = 2 * jj
        b = a + 1
        @pl.when(jj > 0)
        def _():
            pltpu.make_async_copy(emb_q.at[pl.ds(0, CHUNK)], rows_b, sem_sb).wait()
        pltpu.sync_copy(ed.at[c, s, b], idx_b)
        pltpu.async_copy(emb_q.at[idx_b.at[1]], rows_b, sem_gb)
        pltpu.make_async_copy(emb_q.at[pl.ds(0, CHUNK)], rows_a, sem_ga).wait()
        pltpu.async_copy(rows_a, shf.at[idx_a.at[0]], sem_sa, add=True)
        ones_scatter(jj, idx_a, sem_oa)
        pltpu.make_async_copy(emb_q.at[pl.ds(0, CHUNK)], rows_a, sem_sa).wait()

        @pl.when(jj + 1 < NJJ)
        def _():
            pltpu.sync_copy(ed.at[c, s, a + 2], idx_a)
            pltpu.async_copy(emb_q.at[idx_a.at[1]], rows_a, sem_ga)
        pltpu.make_async_copy(emb_q.at[pl.ds(0, CHUNK)], rows_b, sem_gb).wait()
        pltpu.async_copy(rows_b, shf.at[idx_b.at[0]], sem_sb, add=True)
        ones_scatter(jj, idx_b, sem_ob)
        return carry

    lax.fori_loop(0, NJJ, body, 0)
    pltpu.make_async_copy(emb_q.at[pl.ds(0, CHUNK)], rows_b, sem_sb).wait()
    pltpu.make_async_copy(ones_h, ones_v, sem_oa).wait()
    pltpu.make_async_copy(ones_h, ones_v, sem_ob).wait()
    plsc.subcore_barrier()
    pltpu.sync_copy(
        shf.at[pl.ds(s * ROWS_PER_TILE, ROWS_PER_TILE)],
        feat.at[c, pl.ds(s * ROWS_PER_TILE, ROWS_PER_TILE)],
    )
    pltpu.sync_copy(
        shc.at[pl.ds(s * ROWS_PER_TILE, ROWS_PER_TILE)],
        cnt.at[c, pl.ds(s * ROWS_PER_TILE, ROWS_PER_TILE)],
    )


_sc_scatter_call = functools.partial(
    pl.kernel,
    out_type=(
        jax.ShapeDtypeStruct((NC, N_NODES, IN_DIM), jnp.int16),
        jax.ShapeDtypeStruct((NC, N_NODES, CW), jnp.float32),
    ),
    mesh=plsc.VectorSubcoreMesh(core_axis_name="c", subcore_axis_name="s"),
    scratch_types=[
        pltpu.VMEM((2, CHUNK), jnp.int32),
        pltpu.VMEM((2, CHUNK), jnp.int32),
        pltpu.VMEM((CHUNK, IN_DIM), jnp.int16),
        pltpu.VMEM((CHUNK, IN_DIM), jnp.int16),
        pltpu.VMEM((CHUNK, CW), jnp.float32),
        pltpu.VMEM_SHARED((N_NODES, IN_DIM), jnp.int16),
        pltpu.VMEM_SHARED((N_NODES, CW), jnp.float32),
        pltpu.SemaphoreType.DMA,
        pltpu.SemaphoreType.DMA,
        pltpu.SemaphoreType.DMA,
        pltpu.SemaphoreType.DMA,
        pltpu.SemaphoreType.DMA,
        pltpu.SemaphoreType.DMA,
    ],
    compiler_params=pltpu.CompilerParams(use_tc_tiling_on_sc=False),
)(_sc_scatter)


def _tc_body(x_ref, f_ref, c_ref, ws_ref, wne_ref, wno_ref, b_ref, o_ref):
    x = x_ref[...]
    fw = f_ref[...]
    lo = lax.shift_right_arithmetic(lax.shift_left(fw, 16), 16)
    hi = lax.shift_right_arithmetic(fw, 16)
    slo = (lo[0] + lo[1]).astype(jnp.float32)
    shi = (hi[0] + hi[1]).astype(jnp.float32)
    cb = c_ref[...]
    cnt = cb[0, :, :1] + cb[1, :, :1]
    mask = cnt > 0.0
    safe = jnp.where(mask, cnt * QSCALE, QSCALE)
    mlo = jnp.where(mask, slo / safe, 0.0)
    mhi = jnp.where(mask, shi / safe, 0.0)
    acc = jnp.dot(x, ws_ref[...], preferred_element_type=jnp.float32)
    acc += jnp.dot(mlo, wne_ref[...], preferred_element_type=jnp.float32)
    acc += jnp.dot(mhi, wno_ref[...], preferred_element_type=jnp.float32)
    acc += b_ref[...]
    acc = jnp.maximum(acc, 0.0)
    nrm = jnp.sqrt(jnp.sum(acc * acc, axis=1, keepdims=True)) + 1e-9
    o_ref[...] = acc / nrm


def _tc_call(x, featw, cnt, ws, wne, wno, b):
    R = 1000
    grid = (N_NODES // R,)
    return pl.pallas_call(
        _tc_body,
        grid=grid,
        in_specs=[
            pl.BlockSpec((R, IN_DIM), lambda i: (i, 0)),
            pl.BlockSpec((NC, R, IN_DIM // 2), lambda i: (0, i, 0)),
            pl.BlockSpec((NC, R, CW), lambda i: (0, i, 0)),
            pl.BlockSpec((IN_DIM, HID_DIM), lambda i: (0, 0)),
            pl.BlockSpec((IN_DIM // 2, HID_DIM), lambda i: (0, 0)),
            pl.BlockSpec((IN_DIM // 2, HID_DIM), lambda i: (0, 0)),
            pl.BlockSpec((1, HID_DIM), lambda i: (0, 0)),
        ],
        out_specs=pl.BlockSpec((R, HID_DIM), lambda i: (i, 0)),
        out_shape=jax.ShapeDtypeStruct((N_NODES, HID_DIM), jnp.float32),
    )(x, featw, cnt, ws, wne, wno, b)


@jax.jit
def kernel(item_emb, edges, w_self_W, w_self_b, w_neigh_W, w_neigh_b):
    f32 = jnp.float32
    src = edges[:, 0].astype(jnp.int32)
    dst = edges[:, 1].astype(jnp.int32)
    emb_q = jnp.rint(item_emb * QSCALE).astype(jnp.int16)
    ed = jnp.concatenate([
        src.reshape(NC, NS, NCHUNK, 1, CHUNK),
        dst.reshape(NC, NS, NCHUNK, 1, CHUNK),
    ], axis=3)
    ones_h = jnp.ones((CHUNK, CW), f32)
    zf = jnp.zeros((ROWS_PER_TILE, IN_DIM), jnp.int16)
    zc = jnp.zeros((ROWS_PER_TILE, CW), f32)

    feat, cnt = _sc_scatter_call(emb_q, ed, ones_h, zf, zc)
    featw = lax.bitcast_convert_type(
        feat.reshape(NC, N_NODES, IN_DIM // 2, 2), jnp.int32)

    bias = (w_self_b + w_neigh_b).reshape(1, HID_DIM)
    return _tc_call(item_emb, featw, cnt, w_self_W,
                    w_neigh_W[0::2], w_neigh_W[1::2], bias)
